# baseline (speedup 1.0000x reference)
import functools

import jax
import jax.numpy as jnp
from jax import lax
from jax.experimental import pallas as pl
from jax.experimental.pallas import tpu as pltpu
from jax.experimental.pallas import tpu_sc as plsc

BATCH = 16384
SEQ = 20
DIM = 16
VOCAB = 1000000
NC = 2
NS = 16
NW = NC * NS
BPW = BATCH // NW
CB = 4
NCHUNK = BPW // CB
IDX_PER_GATHER = CB * SEQ


def _make_sc_body(nf):
    def body(*refs):
        frs = refs[:nf]
        tab = refs[nf]
        wall = refs[nf + 1]
        out_hbm = refs[nf + 2]
        idxs = refs[nf + 3:2 * nf + 3]
        wv, outv, accbuf = refs[2 * nf + 3:2 * nf + 6]
        bufs = refs[2 * nf + 6:3 * nf + 6]
        sems = refs[3 * nf + 6:3 * nf + 8]

        wid = lax.axis_index("s") * NC + lax.axis_index("c")

        for fr, idx in zip(frs, idxs):
            pltpu.sync_copy(fr.at[wid], idx)
        pltpu.sync_copy(wall, wv)

        def fire(c, p):
            for idx, buf in zip(idxs, bufs):
                pltpu.async_copy(tab.at[idx.at[c]], buf.at[p], sems[p])

        def drain(p):
            for buf in bufs:
                pltpu.make_async_copy(
                    tab.at[pl.ds(0, IDX_PER_GATHER)], buf.at[p], sems[p]).wait()

        def compute(c, p):
            accs = [jnp.zeros((DIM,), jnp.float32) for _ in range(CB)]
            for fi, buf in enumerate(bufs):
                r = buf.at[p]
                for s in range(SEQ):
                    w = wv[fi * SEQ + s]
                    for i in range(CB):
                        accs[i] = accs[i] + r[i * SEQ + s] * w
            for i in range(CB):
                accbuf[pl.ds((c * CB + i) * DIM, DIM)] = accs[i]

        fire(0, 0)
        fire(1, 1)

        def step(k, carry):
            c0 = 2 * k
            drain(0)
            compute(c0, 0)

            @pl.when(c0 + 2 < NCHUNK)
            def _():
                fire(c0 + 2, 0)

            drain(1)
            compute(c0 + 1, 1)

            @pl.when(c0 + 3 < NCHUNK)
            def _():
                fire(c0 + 3, 1)

            return carry

        lax.fori_loop(0, NCHUNK // 2, step, 0)

        lanes = lax.iota(jnp.int32, 16)

        def hsum(v):
            for k in (8, 4, 2, 1):
                v = v + jnp.take(v, lanes ^ k)
            return v

        def fin(g, carry):
            base = g * 16
            out_vec = jnp.zeros((16,), jnp.float32)
            for i in range(16):
                v = accbuf[pl.ds((base + i) * DIM, DIM)]
                out_vec = jnp.where(lanes == i, hsum(v), out_vec)
            outv[pl.ds(base, 16)] = out_vec
            return carry

        lax.fori_loop(0, BPW // 16, fin, 0)

        pltpu.sync_copy(outv, out_hbm.at[pl.ds(wid * BPW, BPW)])

    return body


@functools.lru_cache(maxsize=2)
def _build_sc_kernel(nf):
    scratch = (
        [pltpu.VMEM((NCHUNK, IDX_PER_GATHER), jnp.int32)] * nf
        + [pltpu.VMEM((nf * SEQ, DIM), jnp.float32),
           pltpu.VMEM((BPW,), jnp.float32),
           pltpu.VMEM((BPW * DIM,), jnp.float32)]
        + [pltpu.VMEM((2, IDX_PER_GATHER, DIM), jnp.float32)] * nf
        + [pltpu.SemaphoreType.DMA, pltpu.SemaphoreType.DMA]
    )
    return functools.partial(
        pl.kernel,
        out_type=jax.ShapeDtypeStruct((BATCH,), jnp.float32),
        mesh=plsc.VectorSubcoreMesh(core_axis_name="c", subcore_axis_name="s",
                                    num_cores=NC, num_subcores=NS),
        scratch_types=scratch,
        compiler_params=pltpu.CompilerParams(use_tc_tiling_on_sc=False),
    )(_make_sc_body(nf))


CBLK = 8192
NBLK = (VOCAB + CBLK - 1) // CBLK
MROWS = CBLK // 8
PADROWS = NBLK * MROWS


def _tc_transpose(tt):
    def body(t_ref, o_ref):
        qv = lax.broadcasted_iota(jnp.int32, (128, 128), 0)
        cv = lax.broadcasted_iota(jnp.int32, (128, 128), 1)
        eb = jnp.where(cv == qv, 1.0, 0.0)
        x = t_ref[...]
        xcat = jnp.concatenate(
            [x[:, j * MROWS:(j + 1) * MROWS] for j in range(8)], axis=0)
        o_ref[...] = lax.dot_general(
            xcat, eb, (((0,), (0,)), ((), ())),
            preferred_element_type=jnp.float32)

    return pl.pallas_call(
        body,
        grid=(NBLK,),
        in_specs=[pl.BlockSpec((16, CBLK), lambda i: (0, i))],
        out_specs=pl.BlockSpec((MROWS, 128), lambda i: (i, 0)),
        out_shape=jax.ShapeDtypeStruct((PADROWS, 128), jnp.float32),
        compiler_params=pltpu.CompilerParams(fuse_transposed_lhs_in_matmul=True),
    )(tt)


def _perm(f):
    return (MROWS * (f // CBLK) + f % MROWS) * 8 + (f // MROWS) % 8


def kernel(f1, f2, f3, emb1_table, emb2_table, dense_w, dense_b):
    f1r = _perm(f1).reshape(NW, NCHUNK, IDX_PER_GATHER)
    f2r = _perm(f2).reshape(NW, NCHUNK, IDX_PER_GATHER)
    f3r = _perm(f3).reshape(NW, NCHUNK, IDX_PER_GATHER)
    wperm = dense_w.reshape(SEQ, 3, DIM).transpose(1, 0, 2).reshape(3 * SEQ, DIM)
    o1 = _tc_transpose(emb1_table.T)
    o2 = _tc_transpose(emb2_table.T)
    t1lin = o1.reshape(PADROWS * 8, DIM)
    t2lin = o2.reshape(PADROWS * 8, DIM)
    out12 = _build_sc_kernel(2)(f1r, f2r, t1lin, wperm[:2 * SEQ])
    out3 = _build_sc_kernel(1)(f3r, t2lin, wperm[2 * SEQ:])
    return (out12 + out3).reshape(BATCH, 1) + dense_b


# speedup vs baseline: 1.3219x; 1.3219x over previous
import functools

import jax
import jax.numpy as jnp
from jax import lax
from jax.experimental import pallas as pl
from jax.experimental.pallas import tpu as pltpu
from jax.experimental.pallas import tpu_sc as plsc

BATCH = 16384
SEQ = 20
DIM = 16
VOCAB = 1000000
NC = 2
NS = 16
NW = NC * NS
BPW = BATCH // NW
CB = 4
NCHUNK = BPW // CB
IDX_PER_GATHER = CB * SEQ


def _make_sc_body(nf):
    def body(*refs):
        frs = refs[:nf]
        tab = refs[nf]
        wall = refs[nf + 1]
        out_hbm = refs[nf + 2]
        idxs = refs[nf + 3:2 * nf + 3]
        wv, outv, accbuf = refs[2 * nf + 3:2 * nf + 6]
        bufs = refs[2 * nf + 6:3 * nf + 6]
        sems = refs[3 * nf + 6:3 * nf + 8]

        wid = lax.axis_index("s") * NC + lax.axis_index("c")

        for fr, idx in zip(frs, idxs):
            pltpu.sync_copy(fr.at[wid], idx)
        pltpu.sync_copy(wall, wv)

        def fire(c, p):
            for idx, buf in zip(idxs, bufs):
                pltpu.async_copy(tab.at[idx.at[c]], buf.at[p], sems[p])

        def drain(p):
            for buf in bufs:
                pltpu.make_async_copy(
                    tab.at[pl.ds(0, IDX_PER_GATHER)], buf.at[p], sems[p]).wait()

        def compute(c, p):
            accs = [jnp.zeros((DIM,), jnp.float32) for _ in range(CB)]
            for fi, buf in enumerate(bufs):
                r = buf.at[p]
                for s in range(SEQ):
                    w = wv[fi * SEQ + s]
                    for i in range(CB):
                        accs[i] = accs[i] + r[i * SEQ + s] * w
            for i in range(CB):
                accbuf[pl.ds((c * CB + i) * DIM, DIM)] = accs[i]

        fire(0, 0)
        fire(1, 1)

        def step(k, carry):
            c0 = 2 * k
            drain(0)
            compute(c0, 0)

            @pl.when(c0 + 2 < NCHUNK)
            def _():
                fire(c0 + 2, 0)

            drain(1)
            compute(c0 + 1, 1)

            @pl.when(c0 + 3 < NCHUNK)
            def _():
                fire(c0 + 3, 1)

            return carry

        lax.fori_loop(0, NCHUNK // 2, step, 0)

        lanes = lax.iota(jnp.int32, 16)

        def hsum(v):
            for k in (8, 4, 2, 1):
                v = v + jnp.take(v, lanes ^ k)
            return v

        def fin(g, carry):
            base = g * 16
            out_vec = jnp.zeros((16,), jnp.float32)
            for i in range(16):
                v = accbuf[pl.ds((base + i) * DIM, DIM)]
                out_vec = jnp.where(lanes == i, hsum(v), out_vec)
            outv[pl.ds(base, 16)] = out_vec
            return carry

        lax.fori_loop(0, BPW // 16, fin, 0)

        pltpu.sync_copy(outv, out_hbm.at[pl.ds(wid * BPW, BPW)])

    return body


@functools.lru_cache(maxsize=2)
def _build_sc_kernel(nf):
    scratch = (
        [pltpu.VMEM((NCHUNK, IDX_PER_GATHER), jnp.int32)] * nf
        + [pltpu.VMEM((nf * SEQ, DIM), jnp.float32),
           pltpu.VMEM((BPW,), jnp.float32),
           pltpu.VMEM((BPW * DIM,), jnp.float32)]
        + [pltpu.VMEM((2, IDX_PER_GATHER, DIM), jnp.float32)] * nf
        + [pltpu.SemaphoreType.DMA, pltpu.SemaphoreType.DMA]
    )
    return functools.partial(
        pl.kernel,
        out_type=jax.ShapeDtypeStruct((BATCH,), jnp.float32),
        mesh=plsc.VectorSubcoreMesh(core_axis_name="c", subcore_axis_name="s",
                                    num_cores=NC, num_subcores=NS),
        scratch_types=scratch,
        compiler_params=pltpu.CompilerParams(use_tc_tiling_on_sc=False),
    )(_make_sc_body(nf))


CBLK = 32768
NBLK = (VOCAB + CBLK - 1) // CBLK
MROWS = CBLK // 8
PADROWS = NBLK * MROWS


def _tc_transpose(tt):
    def body(t_ref, o_ref):
        qv = lax.broadcasted_iota(jnp.int32, (128, 128), 0)
        cv = lax.broadcasted_iota(jnp.int32, (128, 128), 1)
        eb = jnp.where(cv == qv, 1.0, 0.0)
        x = t_ref[...]
        xcat = jnp.concatenate(
            [x[:, j * MROWS:(j + 1) * MROWS] for j in range(8)], axis=0)
        o_ref[...] = lax.dot_general(
            xcat, eb, (((0,), (0,)), ((), ())),
            preferred_element_type=jnp.float32)

    return pl.pallas_call(
        body,
        grid=(NBLK,),
        in_specs=[pl.BlockSpec((16, CBLK), lambda i: (0, i))],
        out_specs=pl.BlockSpec((MROWS, 128), lambda i: (i, 0)),
        out_shape=jax.ShapeDtypeStruct((PADROWS, 128), jnp.float32),
        compiler_params=pltpu.CompilerParams(fuse_transposed_lhs_in_matmul=True),
    )(tt)


def _perm(f):
    return (MROWS * (f // CBLK) + f % MROWS) * 8 + (f // MROWS) % 8


def kernel(f1, f2, f3, emb1_table, emb2_table, dense_w, dense_b):
    f1r = _perm(f1).reshape(NW, NCHUNK, IDX_PER_GATHER)
    f2r = _perm(f2).reshape(NW, NCHUNK, IDX_PER_GATHER)
    f3r = _perm(f3).reshape(NW, NCHUNK, IDX_PER_GATHER)
    wperm = dense_w.reshape(SEQ, 3, DIM).transpose(1, 0, 2).reshape(3 * SEQ, DIM)
    o1 = _tc_transpose(emb1_table.T)
    o2 = _tc_transpose(emb2_table.T)
    t1lin = o1.reshape(PADROWS * 8, DIM)
    t2lin = o2.reshape(PADROWS * 8, DIM)
    out12 = _build_sc_kernel(2)(f1r, f2r, t1lin, wperm[:2 * SEQ])
    out3 = _build_sc_kernel(1)(f3r, t2lin, wperm[2 * SEQ:])
    return (out12 + out3).reshape(BATCH, 1) + dense_b


# speedup vs baseline: 1.3301x; 1.0062x over previous
import functools

import jax
import jax.numpy as jnp
from jax import lax
from jax.experimental import pallas as pl
from jax.experimental.pallas import tpu as pltpu
from jax.experimental.pallas import tpu_sc as plsc

BATCH = 16384
SEQ = 20
DIM = 16
VOCAB = 1000000
NC = 2
NS = 16
NW = NC * NS
BPW = BATCH // NW
CB = 4
NCHUNK = BPW // CB
IDX_PER_GATHER = CB * SEQ


def _make_sc_body(nf):
    def body(*refs):
        frs = refs[:nf]
        tab = refs[nf]
        wall = refs[nf + 1]
        out_hbm = refs[nf + 2]
        idxs = refs[nf + 3:2 * nf + 3]
        wv, outv, accbuf = refs[2 * nf + 3:2 * nf + 6]
        bufs = refs[2 * nf + 6:3 * nf + 6]
        sems = refs[3 * nf + 6:3 * nf + 8]

        wid = lax.axis_index("s") * NC + lax.axis_index("c")

        for fr, idx in zip(frs, idxs):
            pltpu.sync_copy(fr.at[wid], idx)
        pltpu.sync_copy(wall, wv)

        def fire(c, p):
            for idx, buf in zip(idxs, bufs):
                pltpu.async_copy(tab.at[idx.at[c]], buf.at[p], sems[p])

        def drain(p):
            for buf in bufs:
                pltpu.make_async_copy(
                    tab.at[pl.ds(0, IDX_PER_GATHER)], buf.at[p], sems[p]).wait()

        def compute(c, p):
            accs = [jnp.zeros((DIM,), jnp.float32) for _ in range(CB)]
            for fi, buf in enumerate(bufs):
                r = buf.at[p]
                for s in range(SEQ):
                    w = wv[fi * SEQ + s]
                    for i in range(CB):
                        accs[i] = accs[i] + r[i * SEQ + s] * w
            for i in range(CB):
                accbuf[pl.ds((c * CB + i) * DIM, DIM)] = accs[i]

        fire(0, 0)
        fire(1, 1)

        def step(k, carry):
            c0 = 2 * k
            drain(0)
            compute(c0, 0)

            @pl.when(c0 + 2 < NCHUNK)
            def _():
                fire(c0 + 2, 0)

            drain(1)
            compute(c0 + 1, 1)

            @pl.when(c0 + 3 < NCHUNK)
            def _():
                fire(c0 + 3, 1)

            return carry

        lax.fori_loop(0, NCHUNK // 2, step, 0)

        lanes = lax.iota(jnp.int32, 16)

        def hsum(v):
            for k in (8, 4, 2, 1):
                v = v + jnp.take(v, lanes ^ k)
            return v

        def fin(g, carry):
            base = g * 16
            out_vec = jnp.zeros((16,), jnp.float32)
            for i in range(16):
                v = accbuf[pl.ds((base + i) * DIM, DIM)]
                out_vec = jnp.where(lanes == i, hsum(v), out_vec)
            outv[pl.ds(base, 16)] = out_vec
            return carry

        lax.fori_loop(0, BPW // 16, fin, 0)

        pltpu.sync_copy(outv, out_hbm.at[pl.ds(wid * BPW, BPW)])

    return body


@functools.lru_cache(maxsize=2)
def _build_sc_kernel(nf):
    scratch = (
        [pltpu.VMEM((NCHUNK, IDX_PER_GATHER), jnp.int32)] * nf
        + [pltpu.VMEM((nf * SEQ, DIM), jnp.float32),
           pltpu.VMEM((BPW,), jnp.float32),
           pltpu.VMEM((BPW * DIM,), jnp.float32)]
        + [pltpu.VMEM((2, IDX_PER_GATHER, DIM), jnp.float32)] * nf
        + [pltpu.SemaphoreType.DMA, pltpu.SemaphoreType.DMA]
    )
    return functools.partial(
        pl.kernel,
        out_type=jax.ShapeDtypeStruct((BATCH,), jnp.float32),
        mesh=plsc.VectorSubcoreMesh(core_axis_name="c", subcore_axis_name="s",
                                    num_cores=NC, num_subcores=NS),
        scratch_types=scratch,
        compiler_params=pltpu.CompilerParams(use_tc_tiling_on_sc=False),
    )(_make_sc_body(nf))


CBLK = 65536
NBLK = (VOCAB + CBLK - 1) // CBLK
MROWS = CBLK // 8
PADROWS = NBLK * MROWS


def _tc_transpose(tt):
    def body(t_ref, o_ref):
        qv = lax.broadcasted_iota(jnp.int32, (128, 128), 0)
        cv = lax.broadcasted_iota(jnp.int32, (128, 128), 1)
        eb = jnp.where(cv == qv, 1.0, 0.0)
        x = t_ref[...]
        xcat = jnp.concatenate(
            [x[:, j * MROWS:(j + 1) * MROWS] for j in range(8)], axis=0)
        o_ref[...] = lax.dot_general(
            xcat, eb, (((0,), (0,)), ((), ())),
            preferred_element_type=jnp.float32)

    return pl.pallas_call(
        body,
        grid=(NBLK,),
        in_specs=[pl.BlockSpec((16, CBLK), lambda i: (0, i))],
        out_specs=pl.BlockSpec((MROWS, 128), lambda i: (i, 0)),
        out_shape=jax.ShapeDtypeStruct((PADROWS, 128), jnp.float32),
        compiler_params=pltpu.CompilerParams(fuse_transposed_lhs_in_matmul=True),
    )(tt)


def _perm(f):
    return (MROWS * (f // CBLK) + f % MROWS) * 8 + (f // MROWS) % 8


def kernel(f1, f2, f3, emb1_table, emb2_table, dense_w, dense_b):
    f1r = _perm(f1).reshape(NW, NCHUNK, IDX_PER_GATHER)
    f2r = _perm(f2).reshape(NW, NCHUNK, IDX_PER_GATHER)
    f3r = _perm(f3).reshape(NW, NCHUNK, IDX_PER_GATHER)
    wperm = dense_w.reshape(SEQ, 3, DIM).transpose(1, 0, 2).reshape(3 * SEQ, DIM)
    o1 = _tc_transpose(emb1_table.T)
    o2 = _tc_transpose(emb2_table.T)
    t1lin = o1.reshape(PADROWS * 8, DIM)
    t2lin = o2.reshape(PADROWS * 8, DIM)
    out12 = _build_sc_kernel(2)(f1r, f2r, t1lin, wperm[:2 * SEQ])
    out3 = _build_sc_kernel(1)(f3r, t2lin, wperm[2 * SEQ:])
    return (out12 + out3).reshape(BATCH, 1) + dense_b
